# baseline (speedup 1.0000x reference)
import jax
import jax.numpy as jnp
from jax.experimental import pallas as pl
from jax.experimental.pallas import tpu as pltpu


def _body(x0_ref, x1_ref, p0_ref, p1_ref, o_ref):
    h = x0_ref.shape[-1]
    o_ref[:, :, :h] = x0_ref[...] + p0_ref[...]
    o_ref[:, :, h:] = x1_ref[...] + p1_ref[...]


def kernel(x, pos_table):
    b, s, d = x.shape
    sb = 2048
    h = d // 2
    grid = (s // sb, b)
    return pl.pallas_call(
        _body,
        grid=grid,
        in_specs=[
            pl.BlockSpec((1, sb, h), lambda i, j: (j, i, 0)),
            pl.BlockSpec((1, sb, h), lambda i, j: (j, i, 1)),
            pl.BlockSpec((sb, h), lambda i, j: (i, 0)),
            pl.BlockSpec((sb, h), lambda i, j: (i, 1)),
        ],
        out_specs=pl.BlockSpec((1, sb, d), lambda i, j: (j, i, 0)),
        out_shape=jax.ShapeDtypeStruct((b, s, d), x.dtype),
        compiler_params=pltpu.CompilerParams(
            dimension_semantics=("parallel", "parallel"),
        ),
    )(x, x, pos_table, pos_table)
